# Initial kernel scaffold; baseline (speedup 1.0000x reference)
import jax
import jax.numpy as jnp
from jax.experimental import pallas as pl


def kernel(x, router_logits, w13, w2):
    raise NotImplementedError("write your pallas kernel here")


# speedup vs baseline: 1.1484x; 1.1484x over previous
import functools

import jax
import jax.numpy as jnp
from jax.experimental import pallas as pl
from jax.experimental.pallas import tpu as pltpu

E = 8
TOP_K = 2
D_MODEL = 2048
D_FF = 1024
BM = 256


def _combine_weights(logits):
    ids = jax.lax.broadcasted_iota(jnp.int32, logits.shape, 1)
    m1 = jnp.max(logits, axis=-1, keepdims=True)
    idx1 = jnp.min(jnp.where(logits == m1, ids, E), axis=-1, keepdims=True)
    l2 = jnp.where(ids == idx1, -jnp.inf, logits)
    m2 = jnp.max(l2, axis=-1, keepdims=True)
    idx2 = jnp.min(jnp.where(l2 == m2, ids, E), axis=-1, keepdims=True)
    e2 = jnp.exp(m2 - m1)
    w1 = 1.0 / (1.0 + e2)
    w2 = 1.0 - w1
    return jnp.where(ids == idx1, w1, 0.0) + jnp.where(ids == idx2, w2, 0.0)


def _moe_body(logits_ref, x_ref, w13_ref, w2_ref, out_ref):
    e = pl.program_id(1)
    logits = logits_ref[...]
    combine = _combine_weights(logits)
    ids = jax.lax.broadcasted_iota(jnp.int32, logits.shape, 1)
    cw = jnp.sum(jnp.where(ids == e, combine, 0.0), axis=-1)

    xb = x_ref[...].astype(jnp.bfloat16)
    gu = jnp.dot(xb, w13_ref[0], preferred_element_type=jnp.float32)
    gate = gu[:, :D_FF]
    up = gu[:, D_FF:]
    h = (gate * jax.nn.sigmoid(gate)) * up
    y = jnp.dot(h.astype(jnp.bfloat16), w2_ref[0],
                preferred_element_type=jnp.float32)
    y = y * cw[:, None]

    @pl.when(e == 0)
    def _():
        out_ref[...] = y

    @pl.when(e != 0)
    def _():
        out_ref[...] += y


def kernel(x, router_logits, w13, w2):
    T = x.shape[0]
    w13b = w13.astype(jnp.bfloat16)
    w2b = w2.astype(jnp.bfloat16)
    grid = (T // BM, E)
    return pl.pallas_call(
        _moe_body,
        grid=grid,
        in_specs=[
            pl.BlockSpec((BM, E), lambda t, e: (t, 0)),
            pl.BlockSpec((BM, D_MODEL), lambda t, e: (t, 0)),
            pl.BlockSpec((1, D_MODEL, 2 * D_FF), lambda t, e: (e, 0, 0)),
            pl.BlockSpec((1, D_FF, D_MODEL), lambda t, e: (e, 0, 0)),
        ],
        out_specs=pl.BlockSpec((BM, D_MODEL), lambda t, e: (t, 0)),
        out_shape=jax.ShapeDtypeStruct((T, D_MODEL), jnp.float32),
        compiler_params=pltpu.CompilerParams(
            dimension_semantics=("arbitrary", "arbitrary"),
        ),
    )(router_logits, x, w13b, w2b)
